# Initial kernel scaffold; baseline (speedup 1.0000x reference)
import jax
import jax.numpy as jnp
from jax.experimental import pallas as pl


def kernel(x, edge_index, true_pairs, negative_pairs, Wl1, Wr1, att1, b1, Wl2, Wr2, att2, b2, cW1, cb1, cW2, cb2):
    raise NotImplementedError("write your pallas kernel here")


# speedup vs baseline: 6.5258x; 6.5258x over previous
import functools

import jax
import jax.numpy as jnp
from jax import lax
from jax.experimental import pallas as pl
from jax.experimental.pallas import tpu as pltpu
from jax.experimental.pallas import tpu_sc as plsc

NC = 2
NS = 16
NW = NC * NS
EB = 64
D = 128


def _mm2_body(x_ref, wl_ref, wr_ref, xl_ref, xr_ref):
    x = x_ref[...]
    xl_ref[...] = jnp.dot(x, wl_ref[...], preferred_element_type=jnp.float32)
    xr_ref[...] = jnp.dot(x, wr_ref[...], preferred_element_type=jnp.float32)


def _transform(x, wl, wr, blk=256):
    n = x.shape[0]
    grid = n // blk
    return pl.pallas_call(
        _mm2_body,
        grid=(grid,),
        in_specs=[
            pl.BlockSpec((blk, D), lambda i: (i, 0)),
            pl.BlockSpec((D, D), lambda i: (0, 0)),
            pl.BlockSpec((D, D), lambda i: (0, 0)),
        ],
        out_specs=[
            pl.BlockSpec((blk, D), lambda i: (i, 0)),
            pl.BlockSpec((blk, D), lambda i: (i, 0)),
        ],
        out_shape=[
            jax.ShapeDtypeStruct((n, D), jnp.float32),
            jax.ShapeDtypeStruct((n, D), jnp.float32),
        ],
    )(x, wl, wr)


def _combine_mm2_body(num_ref, den_ref, b_ref, wl_ref, wr_ref, xl_ref, xr_ref):
    n = num_ref[0] + num_ref[1]
    d = jnp.sum(den_ref[...], axis=0)[:, None]
    h = jnp.maximum(n / (d + 1e-16) + b_ref[0], 0.0)
    xl_ref[...] = jnp.dot(h, wl_ref[...], preferred_element_type=jnp.float32)
    xr_ref[...] = jnp.dot(h, wr_ref[...], preferred_element_type=jnp.float32)


def _combine_transform(num, den, b, wl, wr, blk=256):
    n = num.shape[1]
    grid = n // blk
    return pl.pallas_call(
        _combine_mm2_body,
        grid=(grid,),
        in_specs=[
            pl.BlockSpec((NC, blk, D), lambda i: (0, i, 0)),
            pl.BlockSpec((NW, blk), lambda i: (0, i)),
            pl.BlockSpec((1, D), lambda i: (0, 0)),
            pl.BlockSpec((D, D), lambda i: (0, 0)),
            pl.BlockSpec((D, D), lambda i: (0, 0)),
        ],
        out_specs=[
            pl.BlockSpec((blk, D), lambda i: (i, 0)),
            pl.BlockSpec((blk, D), lambda i: (i, 0)),
        ],
        out_shape=[
            jax.ShapeDtypeStruct((n, D), jnp.float32),
            jax.ShapeDtypeStruct((n, D), jnp.float32),
        ],
    )(num, den, b, wl, wr)


def _combine_body(num_ref, den_ref, b_ref, h_ref):
    n = num_ref[0] + num_ref[1]
    d = jnp.sum(den_ref[...], axis=0)[:, None]
    h_ref[...] = n / (d + 1e-16) + b_ref[0]


def _combine(num, den, b, blk=256):
    n = num.shape[1]
    grid = n // blk
    return pl.pallas_call(
        _combine_body,
        grid=(grid,),
        in_specs=[
            pl.BlockSpec((NC, blk, D), lambda i: (0, i, 0)),
            pl.BlockSpec((NW, blk), lambda i: (0, i)),
            pl.BlockSpec((1, D), lambda i: (0, 0)),
        ],
        out_specs=pl.BlockSpec((blk, D), lambda i: (i, 0)),
        out_shape=jax.ShapeDtypeStruct((n, D), jnp.float32),
    )(num, den, b)


def _classifier_body(pos_ref, neg_ref, w1_ref, b1_ref, w2_ref, b2_ref,
                     lp_ref, ln_ref, loss_ref, acc_ref):
    i = pl.program_id(0)
    nsteps = pl.num_programs(0)

    @pl.when(i == 0)
    def _():
        acc_ref[0] = 0.0

    w1 = w1_ref[...]
    w2 = w2_ref[...]
    b1 = b1_ref[0]
    b2 = b2_ref[0, 0]

    zp = jnp.maximum(jnp.dot(pos_ref[...], w1, preferred_element_type=jnp.float32) + b1, 0.0)
    zn = jnp.maximum(jnp.dot(neg_ref[...], w1, preferred_element_type=jnp.float32) + b1, 0.0)
    lop = jnp.dot(zp, w2, preferred_element_type=jnp.float32)[:, 0:1] + b2
    lon = jnp.dot(zn, w2, preferred_element_type=jnp.float32)[:, 0:1] + b2
    sp = jax.nn.sigmoid(lop)
    sn = jax.nn.sigmoid(lon)
    lp_ref[...] = sp
    ln_ref[...] = sn
    pp = jnp.clip(sp, 1e-7, 1.0 - 1e-7)
    pn = jnp.clip(sn, 1e-7, 1.0 - 1e-7)
    acc_ref[0] += jnp.sum(jnp.log(pp)) + jnp.sum(jnp.log(1.0 - pn))

    @pl.when(i == nsteps - 1)
    def _():
        loss_ref[...] = (-acc_ref[0] / (2.0 * ln_ref.shape[0] * nsteps)
                         ) * jnp.ones((1, 1), jnp.float32)


def _classifier(pos, neg, w1, b1, w2pad, b2, blk=512):
    p = pos.shape[0]
    grid = p // blk
    return pl.pallas_call(
        _classifier_body,
        grid=(grid,),
        in_specs=[
            pl.BlockSpec((blk, 2 * D), lambda i: (i, 0)),
            pl.BlockSpec((blk, 2 * D), lambda i: (i, 0)),
            pl.BlockSpec((2 * D, D), lambda i: (0, 0)),
            pl.BlockSpec((1, D), lambda i: (0, 0)),
            pl.BlockSpec((D, D), lambda i: (0, 0)),
            pl.BlockSpec((1, 1), lambda i: (0, 0)),
        ],
        out_specs=[
            pl.BlockSpec((blk, 1), lambda i: (i, 0)),
            pl.BlockSpec((blk, 1), lambda i: (i, 0)),
            pl.BlockSpec((1, 1), lambda i: (0, 0)),
        ],
        out_shape=[
            jax.ShapeDtypeStruct((p, 1), jnp.float32),
            jax.ShapeDtypeStruct((p, 1), jnp.float32),
            jax.ShapeDtypeStruct((1, 1), jnp.float32),
        ],
        scratch_shapes=[pltpu.SMEM((1,), jnp.float32)],
    )(pos, neg, w1, b1, w2pad, b2)


def _edge_pass(xl, xr, src, dst, att, etot):
    n_pad = xl.shape[0]
    ep = src.shape[0]
    tpt = ep // NW
    nb = tpt // EB
    stripe = n_pad // NS

    mesh = plsc.VectorSubcoreMesh(core_axis_name="c", subcore_axis_name="s")

    @functools.partial(
        pl.kernel,
        out_type=(
            jax.ShapeDtypeStruct((NC, n_pad, D), jnp.float32),
            jax.ShapeDtypeStruct((NW, n_pad), jnp.float32),
        ),
        mesh=mesh,
        scratch_types=[
            pltpu.VMEM((EB,), jnp.int32),
            pltpu.VMEM((EB,), jnp.int32),
            pltpu.VMEM((EB, D), jnp.float32),
            pltpu.VMEM((EB, D), jnp.float32),
            pltpu.VMEM((D,), jnp.float32),
            pltpu.VMEM((n_pad,), jnp.float32),
            pltpu.VMEM_SHARED((n_pad, D), jnp.float32),
            pltpu.SemaphoreType.DMA,
            pltpu.SemaphoreType.DMA,
        ],
    )
    def k(xl_h, xr_h, src_h, dst_h, att_h, num_h, den_h,
          siv, div, xlg, xrg, attv, denv, acc, sem0, sem1):
        c = lax.axis_index("c")
        s = lax.axis_index("s")
        wid = s * NC + c

        pltpu.sync_copy(att_h, attv)

        zero16 = jnp.zeros((16,), jnp.float32)
        iota16 = lax.broadcasted_iota(jnp.int32, (16,), 0)
        _dn = lax.GatherDimensionNumbers(
            offset_dims=(), collapsed_slice_dims=(0,), start_index_map=(0,))

        def _shuf(v, idx):
            return lax.gather(v, idx[:, None], _dn, slice_sizes=(1,),
                              mode=lax.GatherScatterMode.PROMISE_IN_BOUNDS)

        def zden(i, carry):
            denv[pl.ds(16 * i, 16)] = zero16
            return carry

        lax.fori_loop(0, n_pad // 16, zden, 0)

        def zrow(e, carry):
            for k8 in range(D // 16):
                xlg[e, pl.ds(16 * k8, 16)] = zero16
            return carry

        lax.fori_loop(0, EB, zrow, 0)
        r0 = s * stripe
        for j in range(stripe // EB):
            pltpu.sync_copy(xlg, acc.at[pl.ds(r0 + j * EB, EB)])
        plsc.subcore_barrier()

        ebase = wid * tpt

        def step(it, carry):
            b = ebase + it * EB
            pltpu.sync_copy(src_h.at[pl.ds(b, EB)], siv)
            pltpu.sync_copy(dst_h.at[pl.ds(b, EB)], div)
            pltpu.async_copy(xl_h.at[siv], xlg, sem0).wait()
            pltpu.async_copy(xr_h.at[div], xrg, sem1).wait()

            def group(g, gcarry):
                dv = div[pl.ds(16 * g, 16)]
                for j in range(16):
                    e = 16 * g + j
                    xs = []
                    a = zero16
                    for k8 in range(D // 16):
                        xv = xlg[e, pl.ds(16 * k8, 16)]
                        xs.append(xv)
                        v = xv + xrg[e, pl.ds(16 * k8, 16)]
                        lv = jnp.maximum(v, 0.2 * v)
                        a = a + lv * attv[pl.ds(16 * k8, 16)]
                    for sh in (1, 2, 4, 8):
                        a = a + _shuf(a, iota16 ^ sh)
                    w = jnp.exp(a)
                    w = jnp.where(b + e < etot, w, 0.0)
                    for k8 in range(D // 16):
                        xlg[e, pl.ds(16 * k8, 16)] = xs[k8] * w
                    dval = dv[j]
                    base = dval & -16
                    lane = dval & 15
                    chunk = denv[pl.ds(base, 16)]
                    denv[pl.ds(base, 16)] = (
                        chunk + jnp.where(iota16 == lane, w, 0.0))
                return gcarry

            lax.fori_loop(0, EB // 16, group, 0)
            pltpu.sync_copy(xlg, acc.at[div], add=True)
            return carry

        lax.fori_loop(0, nb, step, 0)
        plsc.subcore_barrier()

        for j in range(stripe // EB):
            rr = r0 + j * EB
            pltpu.sync_copy(acc.at[pl.ds(rr, EB)], num_h.at[c, pl.ds(rr, EB)])
        pltpu.sync_copy(denv, den_h.at[wid])

    return k(xl, xr, src, dst, att)


def _pair_gather(h, tp0, tp1, np0, np1):
    p = tp0.shape[0]
    rpt = p // NW

    mesh = plsc.VectorSubcoreMesh(core_axis_name="c", subcore_axis_name="s")

    @functools.partial(
        pl.kernel,
        out_type=(
            jax.ShapeDtypeStruct((p, 2 * D), jnp.float32),
            jax.ShapeDtypeStruct((p, 2 * D), jnp.float32),
        ),
        mesh=mesh,
        scratch_types=[
            pltpu.VMEM((rpt,), jnp.int32),
            pltpu.VMEM((rpt, D), jnp.float32),
            pltpu.SemaphoreType.DMA,
        ],
    )
    def k(h_h, tp0_h, tp1_h, np0_h, np1_h, pos_h, neg_h, iv, g, sem):
        c = lax.axis_index("c")
        s = lax.axis_index("s")
        wid = s * NC + c
        base = wid * rpt
        for idx_h, out_h, col in ((tp0_h, pos_h, 0), (tp1_h, pos_h, D),
                                  (np0_h, neg_h, 0), (np1_h, neg_h, D)):
            pltpu.sync_copy(idx_h.at[pl.ds(base, rpt)], iv)
            pltpu.async_copy(h_h.at[iv], g, sem).wait()
            pltpu.sync_copy(g, out_h.at[pl.ds(base, rpt), pl.ds(col, D)])

    return k(h, tp0, tp1, np0, np1)


def kernel(x, edge_index, true_pairs, negative_pairs,
           Wl1, Wr1, att1, b1, Wl2, Wr2, att2, b2, cW1, cb1, cW2, cb2):
    N = x.shape[0]
    E = edge_index.shape[1]
    etot = E + N

    n_pad = -(-N // (NS * EB)) * (NS * EB)
    x_pad = jnp.pad(x, ((0, n_pad - N), (0, 0)))

    ep = -(-etot // (NW * EB)) * (NW * EB)
    loop_idx = jnp.arange(N, dtype=edge_index.dtype)
    padv = jnp.zeros((ep - etot,), edge_index.dtype)
    src = jnp.concatenate([edge_index[0], loop_idx, padv])
    dst = jnp.concatenate([edge_index[1], loop_idx, padv])

    b1r = jnp.reshape(b1, (1, D))
    b2r = jnp.reshape(b2, (1, D))
    cb1r = jnp.reshape(cb1, (1, D))
    cb2r = jnp.reshape(cb2, (1, 1))
    cW2pad = jnp.pad(cW2, ((0, 0), (0, D - cW2.shape[1])))

    xl1, xr1 = _transform(x_pad, Wl1, Wr1)
    num1, den1 = _edge_pass(xl1, xr1, src, dst, att1, etot)
    xl2, xr2 = _combine_transform(num1, den1, b1r, Wl2, Wr2)

    num2, den2 = _edge_pass(xl2, xr2, src, dst, att2, etot)
    h2 = _combine(num2, den2, b2r)

    tp0 = true_pairs[:, 0]
    tp1 = true_pairs[:, 1]
    np0 = negative_pairs[:, 0]
    np1 = negative_pairs[:, 1]
    pos, neg = _pair_gather(h2, tp0, tp1, np0, np1)
    lp, ln, loss = _classifier(pos, neg, cW1, cb1r, cW2pad, cb2r)

    logits = jnp.concatenate([lp, ln], axis=0)
    return (pos, neg, logits, jnp.reshape(loss, ()))


# speedup vs baseline: 9.8849x; 1.5147x over previous
import functools

import jax
import jax.numpy as jnp
from jax import lax
from jax.experimental import pallas as pl
from jax.experimental.pallas import tpu as pltpu
from jax.experimental.pallas import tpu_sc as plsc

NC = 2
NS = 16
NW = NC * NS
EB = 64
D = 128


def _mm2_body(x_ref, wl_ref, wr_ref, xl_ref, xr_ref):
    x = x_ref[...]
    xl_ref[...] = jnp.dot(x, wl_ref[...], preferred_element_type=jnp.float32)
    xr_ref[...] = jnp.dot(x, wr_ref[...], preferred_element_type=jnp.float32)


def _transform(x, wl, wr, blk=256):
    n = x.shape[0]
    grid = n // blk
    return pl.pallas_call(
        _mm2_body,
        grid=(grid,),
        in_specs=[
            pl.BlockSpec((blk, D), lambda i: (i, 0)),
            pl.BlockSpec((D, D), lambda i: (0, 0)),
            pl.BlockSpec((D, D), lambda i: (0, 0)),
        ],
        out_specs=[
            pl.BlockSpec((blk, D), lambda i: (i, 0)),
            pl.BlockSpec((blk, D), lambda i: (i, 0)),
        ],
        out_shape=[
            jax.ShapeDtypeStruct((n, D), jnp.float32),
            jax.ShapeDtypeStruct((n, D), jnp.float32),
        ],
    )(x, wl, wr)


def _combine_mm2_body(num_ref, den_ref, b_ref, wl_ref, wr_ref, xl_ref, xr_ref):
    n = num_ref[0] + num_ref[1]
    d = jnp.sum(den_ref[...], axis=0)[:, None]
    h = jnp.maximum(n / (d + 1e-16) + b_ref[0], 0.0)
    xl_ref[...] = jnp.dot(h, wl_ref[...], preferred_element_type=jnp.float32)
    xr_ref[...] = jnp.dot(h, wr_ref[...], preferred_element_type=jnp.float32)


def _combine_transform(num, den, b, wl, wr, blk=256):
    n = num.shape[1]
    grid = n // blk
    return pl.pallas_call(
        _combine_mm2_body,
        grid=(grid,),
        in_specs=[
            pl.BlockSpec((NC, blk, D), lambda i: (0, i, 0)),
            pl.BlockSpec((NW, blk), lambda i: (0, i)),
            pl.BlockSpec((1, D), lambda i: (0, 0)),
            pl.BlockSpec((D, D), lambda i: (0, 0)),
            pl.BlockSpec((D, D), lambda i: (0, 0)),
        ],
        out_specs=[
            pl.BlockSpec((blk, D), lambda i: (i, 0)),
            pl.BlockSpec((blk, D), lambda i: (i, 0)),
        ],
        out_shape=[
            jax.ShapeDtypeStruct((n, D), jnp.float32),
            jax.ShapeDtypeStruct((n, D), jnp.float32),
        ],
    )(num, den, b, wl, wr)


def _combine_body(num_ref, den_ref, b_ref, h_ref):
    n = num_ref[0] + num_ref[1]
    d = jnp.sum(den_ref[...], axis=0)[:, None]
    h_ref[...] = n / (d + 1e-16) + b_ref[0]


def _combine(num, den, b, blk=256):
    n = num.shape[1]
    grid = n // blk
    return pl.pallas_call(
        _combine_body,
        grid=(grid,),
        in_specs=[
            pl.BlockSpec((NC, blk, D), lambda i: (0, i, 0)),
            pl.BlockSpec((NW, blk), lambda i: (0, i)),
            pl.BlockSpec((1, D), lambda i: (0, 0)),
        ],
        out_specs=pl.BlockSpec((blk, D), lambda i: (i, 0)),
        out_shape=jax.ShapeDtypeStruct((n, D), jnp.float32),
    )(num, den, b)


def _classifier_body(pos_ref, neg_ref, w1_ref, b1_ref, w2_ref, b2_ref,
                     lp_ref, ln_ref, loss_ref, acc_ref):
    i = pl.program_id(0)
    nsteps = pl.num_programs(0)

    @pl.when(i == 0)
    def _():
        acc_ref[0] = 0.0

    w1 = w1_ref[...]
    w2 = w2_ref[...]
    b1 = b1_ref[0]
    b2 = b2_ref[0, 0]

    zp = jnp.maximum(jnp.dot(pos_ref[...], w1, preferred_element_type=jnp.float32) + b1, 0.0)
    zn = jnp.maximum(jnp.dot(neg_ref[...], w1, preferred_element_type=jnp.float32) + b1, 0.0)
    lop = jnp.dot(zp, w2, preferred_element_type=jnp.float32)[:, 0:1] + b2
    lon = jnp.dot(zn, w2, preferred_element_type=jnp.float32)[:, 0:1] + b2
    sp = jax.nn.sigmoid(lop)
    sn = jax.nn.sigmoid(lon)
    lp_ref[...] = sp
    ln_ref[...] = sn
    pp = jnp.clip(sp, 1e-7, 1.0 - 1e-7)
    pn = jnp.clip(sn, 1e-7, 1.0 - 1e-7)
    acc_ref[0] += jnp.sum(jnp.log(pp)) + jnp.sum(jnp.log(1.0 - pn))

    @pl.when(i == nsteps - 1)
    def _():
        loss_ref[...] = (-acc_ref[0] / (2.0 * ln_ref.shape[0] * nsteps)
                         ) * jnp.ones((1, 1), jnp.float32)


def _classifier(pos, neg, w1, b1, w2pad, b2, blk=512):
    p = pos.shape[0]
    grid = p // blk
    return pl.pallas_call(
        _classifier_body,
        grid=(grid,),
        in_specs=[
            pl.BlockSpec((blk, 2 * D), lambda i: (i, 0)),
            pl.BlockSpec((blk, 2 * D), lambda i: (i, 0)),
            pl.BlockSpec((2 * D, D), lambda i: (0, 0)),
            pl.BlockSpec((1, D), lambda i: (0, 0)),
            pl.BlockSpec((D, D), lambda i: (0, 0)),
            pl.BlockSpec((1, 1), lambda i: (0, 0)),
        ],
        out_specs=[
            pl.BlockSpec((blk, 1), lambda i: (i, 0)),
            pl.BlockSpec((blk, 1), lambda i: (i, 0)),
            pl.BlockSpec((1, 1), lambda i: (0, 0)),
        ],
        out_shape=[
            jax.ShapeDtypeStruct((p, 1), jnp.float32),
            jax.ShapeDtypeStruct((p, 1), jnp.float32),
            jax.ShapeDtypeStruct((1, 1), jnp.float32),
        ],
        scratch_shapes=[pltpu.SMEM((1,), jnp.float32)],
    )(pos, neg, w1, b1, w2pad, b2)


def _edge_pass(xl, xr, src, dst, att, etot):
    n_pad = xl.shape[0]
    ep = src.shape[0]
    tpt = ep // NW
    nb = tpt // EB
    stripe = n_pad // NS

    mesh = plsc.VectorSubcoreMesh(core_axis_name="c", subcore_axis_name="s")

    @functools.partial(
        pl.kernel,
        out_type=(
            jax.ShapeDtypeStruct((NC, n_pad, D), jnp.float32),
            jax.ShapeDtypeStruct((NW, n_pad), jnp.float32),
        ),
        mesh=mesh,
        scratch_types=[
            pltpu.VMEM((2, EB), jnp.int32),
            pltpu.VMEM((2, EB), jnp.int32),
            pltpu.VMEM((2, EB, D), jnp.float32),
            pltpu.VMEM((2, EB, D), jnp.float32),
            pltpu.VMEM((D,), jnp.float32),
            pltpu.VMEM((n_pad,), jnp.float32),
            pltpu.VMEM_SHARED((n_pad, D), jnp.float32),
            [pltpu.SemaphoreType.DMA] * 2,
            [pltpu.SemaphoreType.DMA] * 2,
            [pltpu.SemaphoreType.DMA] * 2,
        ],
    )
    def k(xl_h, xr_h, src_h, dst_h, att_h, num_h, den_h,
          siv, div, xlg, xrg, attv, denv, acc, gsl, gsr, ssc):
        c = lax.axis_index("c")
        s = lax.axis_index("s")
        wid = s * NC + c

        pltpu.sync_copy(att_h, attv)

        zero16 = jnp.zeros((16,), jnp.float32)
        iota16 = lax.broadcasted_iota(jnp.int32, (16,), 0)
        _dn = lax.GatherDimensionNumbers(
            offset_dims=(), collapsed_slice_dims=(0,), start_index_map=(0,))

        def _shuf(v, idx):
            return lax.gather(v, idx[:, None], _dn, slice_sizes=(1,),
                              mode=lax.GatherScatterMode.PROMISE_IN_BOUNDS)

        def zden(i, carry):
            denv[pl.ds(16 * i, 16)] = zero16
            return carry

        lax.fori_loop(0, n_pad // 16, zden, 0)

        def zrow(e, carry):
            for k8 in range(D // 16):
                xlg[0, e, pl.ds(16 * k8, 16)] = zero16
            return carry

        lax.fori_loop(0, EB, zrow, 0)
        r0 = s * stripe
        for j in range(stripe // EB):
            pltpu.sync_copy(xlg.at[0], acc.at[pl.ds(r0 + j * EB, EB)])
        plsc.subcore_barrier()

        ebase = wid * tpt
        attc = [attv[pl.ds(16 * k8, 16)] for k8 in range(D // 16)]

        def start_gathers(it, bank):
            b = ebase + it * EB
            pltpu.sync_copy(src_h.at[pl.ds(b, EB)], siv.at[bank])
            pltpu.sync_copy(dst_h.at[pl.ds(b, EB)], div.at[bank])
            pltpu.async_copy(xl_h.at[siv.at[bank]], xlg.at[bank], gsl[bank])
            pltpu.async_copy(xr_h.at[div.at[bank]], xrg.at[bank], gsr[bank])

        start_gathers(0, 0)

        def outer(ot, carry):
            for bank in (0, 1):
                it = 2 * ot + bank
                nxt = it + 1
                other = 1 - bank

                @pl.when(nxt < nb)
                def _():
                    @pl.when(it >= 1)
                    def _():
                        pltpu.make_async_copy(
                            xlg.at[other], acc.at[div.at[other]],
                            ssc[other]).wait()
                    start_gathers(nxt, other)

                pltpu.make_async_copy(
                    xl_h.at[siv.at[bank]], xlg.at[bank], gsl[bank]).wait()
                pltpu.make_async_copy(
                    xr_h.at[div.at[bank]], xrg.at[bank], gsr[bank]).wait()

                b = ebase + it * EB

                def group(g, gcarry):
                    dv = div[bank, pl.ds(16 * g, 16)]
                    for j in range(16):
                        e = 16 * g + j
                        xs = []
                        a = zero16
                        for k8 in range(D // 16):
                            xv = xlg[bank, e, pl.ds(16 * k8, 16)]
                            xs.append(xv)
                            v = xv + xrg[bank, e, pl.ds(16 * k8, 16)]
                            lv = jnp.maximum(v, 0.2 * v)
                            a = a + lv * attc[k8]
                        for sh in (1, 2, 4, 8):
                            a = a + _shuf(a, iota16 ^ sh)
                        w = jnp.exp(a)
                        w = jnp.where(b + e < etot, w, 0.0)
                        for k8 in range(D // 16):
                            xlg[bank, e, pl.ds(16 * k8, 16)] = xs[k8] * w
                        dval = dv[j]
                        base = dval & -16
                        lane = dval & 15
                        chunk = denv[pl.ds(base, 16)]
                        denv[pl.ds(base, 16)] = (
                            chunk + jnp.where(iota16 == lane, w, 0.0))
                    return gcarry

                lax.fori_loop(0, EB // 16, group, 0)
                pltpu.async_copy(xlg.at[bank], acc.at[div.at[bank]],
                                 ssc[bank], add=True)
            return carry

        lax.fori_loop(0, nb // 2, outer, 0)
        for bank in (0, 1):
            pltpu.make_async_copy(
                xlg.at[bank], acc.at[div.at[bank]], ssc[bank]).wait()
        plsc.subcore_barrier()

        for j in range(stripe // EB):
            rr = r0 + j * EB
            pltpu.sync_copy(acc.at[pl.ds(rr, EB)], num_h.at[c, pl.ds(rr, EB)])
        pltpu.sync_copy(denv, den_h.at[wid])

    return k(xl, xr, src, dst, att)


def _pair_gather(h, tp0, tp1, np0, np1):
    p = tp0.shape[0]
    rpt = p // NW

    mesh = plsc.VectorSubcoreMesh(core_axis_name="c", subcore_axis_name="s")

    @functools.partial(
        pl.kernel,
        out_type=(
            jax.ShapeDtypeStruct((p, 2 * D), jnp.float32),
            jax.ShapeDtypeStruct((p, 2 * D), jnp.float32),
        ),
        mesh=mesh,
        scratch_types=[
            pltpu.VMEM((rpt,), jnp.int32),
            pltpu.VMEM((rpt, D), jnp.float32),
            pltpu.SemaphoreType.DMA,
        ],
    )
    def k(h_h, tp0_h, tp1_h, np0_h, np1_h, pos_h, neg_h, iv, g, sem):
        c = lax.axis_index("c")
        s = lax.axis_index("s")
        wid = s * NC + c
        base = wid * rpt
        for idx_h, out_h, col in ((tp0_h, pos_h, 0), (tp1_h, pos_h, D),
                                  (np0_h, neg_h, 0), (np1_h, neg_h, D)):
            pltpu.sync_copy(idx_h.at[pl.ds(base, rpt)], iv)
            pltpu.async_copy(h_h.at[iv], g, sem).wait()
            pltpu.sync_copy(g, out_h.at[pl.ds(base, rpt), pl.ds(col, D)])

    return k(h, tp0, tp1, np0, np1)


def kernel(x, edge_index, true_pairs, negative_pairs,
           Wl1, Wr1, att1, b1, Wl2, Wr2, att2, b2, cW1, cb1, cW2, cb2):
    N = x.shape[0]
    E = edge_index.shape[1]
    etot = E + N

    n_pad = -(-N // (NS * EB)) * (NS * EB)
    x_pad = jnp.pad(x, ((0, n_pad - N), (0, 0)))

    ep = -(-etot // (NW * EB)) * (NW * EB)
    loop_idx = jnp.arange(N, dtype=edge_index.dtype)
    padv = jnp.zeros((ep - etot,), edge_index.dtype)
    src = jnp.concatenate([edge_index[0], loop_idx, padv])
    dst = jnp.concatenate([edge_index[1], loop_idx, padv])

    b1r = jnp.reshape(b1, (1, D))
    b2r = jnp.reshape(b2, (1, D))
    cb1r = jnp.reshape(cb1, (1, D))
    cb2r = jnp.reshape(cb2, (1, 1))
    cW2pad = jnp.pad(cW2, ((0, 0), (0, D - cW2.shape[1])))

    xl1, xr1 = _transform(x_pad, Wl1, Wr1)
    num1, den1 = _edge_pass(xl1, xr1, src, dst, att1, etot)
    xl2, xr2 = _combine_transform(num1, den1, b1r, Wl2, Wr2)

    num2, den2 = _edge_pass(xl2, xr2, src, dst, att2, etot)
    h2 = _combine(num2, den2, b2r)

    tp0 = true_pairs[:, 0]
    tp1 = true_pairs[:, 1]
    np0 = negative_pairs[:, 0]
    np1 = negative_pairs[:, 1]
    pos, neg = _pair_gather(h2, tp0, tp1, np0, np1)
    lp, ln, loss = _classifier(pos, neg, cW1, cb1r, cW2pad, cb2r)

    logits = jnp.concatenate([lp, ln], axis=0)
    return (pos, neg, logits, jnp.reshape(loss, ()))


# speedup vs baseline: 12.2155x; 1.2358x over previous
import functools

import jax
import jax.numpy as jnp
from jax import lax
from jax.experimental import pallas as pl
from jax.experimental.pallas import tpu as pltpu
from jax.experimental.pallas import tpu_sc as plsc

NC = 2
NS = 16
NW = NC * NS
EB = 64
D = 128


def _mm2_body(x_ref, wl_ref, wr_ref, xl_ref, xr_ref):
    x = x_ref[...]
    xl_ref[...] = jnp.dot(x, wl_ref[...], preferred_element_type=jnp.float32)
    xr_ref[...] = jnp.dot(x, wr_ref[...], preferred_element_type=jnp.float32)


def _transform(x, wl, wr, blk=256):
    n = x.shape[0]
    grid = n // blk
    return pl.pallas_call(
        _mm2_body,
        grid=(grid,),
        in_specs=[
            pl.BlockSpec((blk, D), lambda i: (i, 0)),
            pl.BlockSpec((D, D), lambda i: (0, 0)),
            pl.BlockSpec((D, D), lambda i: (0, 0)),
        ],
        out_specs=[
            pl.BlockSpec((blk, D), lambda i: (i, 0)),
            pl.BlockSpec((blk, D), lambda i: (i, 0)),
        ],
        out_shape=[
            jax.ShapeDtypeStruct((n, D), jnp.float32),
            jax.ShapeDtypeStruct((n, D), jnp.float32),
        ],
    )(x, wl, wr)


def _combine_mm2_body(num_ref, den_ref, b_ref, wl_ref, wr_ref, xl_ref, xr_ref):
    n = num_ref[0] + num_ref[1]
    d = jnp.sum(den_ref[...], axis=0)[:, None]
    h = jnp.maximum(n / (d + 1e-16) + b_ref[0], 0.0)
    xl_ref[...] = jnp.dot(h, wl_ref[...], preferred_element_type=jnp.float32)
    xr_ref[...] = jnp.dot(h, wr_ref[...], preferred_element_type=jnp.float32)


def _combine_transform(num, den, b, wl, wr, blk=256):
    n = num.shape[1]
    grid = n // blk
    return pl.pallas_call(
        _combine_mm2_body,
        grid=(grid,),
        in_specs=[
            pl.BlockSpec((NC, blk, D), lambda i: (0, i, 0)),
            pl.BlockSpec((NW, blk), lambda i: (0, i)),
            pl.BlockSpec((1, D), lambda i: (0, 0)),
            pl.BlockSpec((D, D), lambda i: (0, 0)),
            pl.BlockSpec((D, D), lambda i: (0, 0)),
        ],
        out_specs=[
            pl.BlockSpec((blk, D), lambda i: (i, 0)),
            pl.BlockSpec((blk, D), lambda i: (i, 0)),
        ],
        out_shape=[
            jax.ShapeDtypeStruct((n, D), jnp.float32),
            jax.ShapeDtypeStruct((n, D), jnp.float32),
        ],
    )(num, den, b, wl, wr)


def _combine_body(num_ref, den_ref, b_ref, h_ref):
    n = num_ref[0] + num_ref[1]
    d = jnp.sum(den_ref[...], axis=0)[:, None]
    h_ref[...] = n / (d + 1e-16) + b_ref[0]


def _combine(num, den, b, blk=256):
    n = num.shape[1]
    grid = n // blk
    return pl.pallas_call(
        _combine_body,
        grid=(grid,),
        in_specs=[
            pl.BlockSpec((NC, blk, D), lambda i: (0, i, 0)),
            pl.BlockSpec((NW, blk), lambda i: (0, i)),
            pl.BlockSpec((1, D), lambda i: (0, 0)),
        ],
        out_specs=pl.BlockSpec((blk, D), lambda i: (i, 0)),
        out_shape=jax.ShapeDtypeStruct((n, D), jnp.float32),
    )(num, den, b)


def _classifier_body(pos_ref, neg_ref, w1_ref, b1_ref, w2_ref, b2_ref,
                     lp_ref, ln_ref, loss_ref, acc_ref):
    i = pl.program_id(0)
    nsteps = pl.num_programs(0)

    @pl.when(i == 0)
    def _():
        acc_ref[0] = 0.0

    w1 = w1_ref[...]
    w2 = w2_ref[...]
    b1 = b1_ref[0]
    b2 = b2_ref[0, 0]

    zp = jnp.maximum(jnp.dot(pos_ref[...], w1, preferred_element_type=jnp.float32) + b1, 0.0)
    zn = jnp.maximum(jnp.dot(neg_ref[...], w1, preferred_element_type=jnp.float32) + b1, 0.0)
    lop = jnp.dot(zp, w2, preferred_element_type=jnp.float32)[:, 0:1] + b2
    lon = jnp.dot(zn, w2, preferred_element_type=jnp.float32)[:, 0:1] + b2
    sp = jax.nn.sigmoid(lop)
    sn = jax.nn.sigmoid(lon)
    lp_ref[...] = sp
    ln_ref[...] = sn
    pp = jnp.clip(sp, 1e-7, 1.0 - 1e-7)
    pn = jnp.clip(sn, 1e-7, 1.0 - 1e-7)
    acc_ref[0] += jnp.sum(jnp.log(pp)) + jnp.sum(jnp.log(1.0 - pn))

    @pl.when(i == nsteps - 1)
    def _():
        loss_ref[...] = (-acc_ref[0] / (2.0 * ln_ref.shape[0] * nsteps)
                         ) * jnp.ones((1, 1), jnp.float32)


def _classifier(pos, neg, w1, b1, w2pad, b2, blk=512):
    p = pos.shape[0]
    grid = p // blk
    return pl.pallas_call(
        _classifier_body,
        grid=(grid,),
        in_specs=[
            pl.BlockSpec((blk, 2 * D), lambda i: (i, 0)),
            pl.BlockSpec((blk, 2 * D), lambda i: (i, 0)),
            pl.BlockSpec((2 * D, D), lambda i: (0, 0)),
            pl.BlockSpec((1, D), lambda i: (0, 0)),
            pl.BlockSpec((D, D), lambda i: (0, 0)),
            pl.BlockSpec((1, 1), lambda i: (0, 0)),
        ],
        out_specs=[
            pl.BlockSpec((blk, 1), lambda i: (i, 0)),
            pl.BlockSpec((blk, 1), lambda i: (i, 0)),
            pl.BlockSpec((1, 1), lambda i: (0, 0)),
        ],
        out_shape=[
            jax.ShapeDtypeStruct((p, 1), jnp.float32),
            jax.ShapeDtypeStruct((p, 1), jnp.float32),
            jax.ShapeDtypeStruct((1, 1), jnp.float32),
        ],
        scratch_shapes=[pltpu.SMEM((1,), jnp.float32)],
    )(pos, neg, w1, b1, w2pad, b2)


def _edge_pass(xl, xr, src, dst, att, etot):
    n_pad = xl.shape[0]
    ep = src.shape[0]
    tpt = ep // NW
    nb = tpt // EB
    stripe = n_pad // NS

    mesh = plsc.VectorSubcoreMesh(core_axis_name="c", subcore_axis_name="s")

    @functools.partial(
        pl.kernel,
        out_type=(
            jax.ShapeDtypeStruct((NC, n_pad, D), jnp.float32),
            jax.ShapeDtypeStruct((NW, n_pad), jnp.float32),
        ),
        mesh=mesh,
        scratch_types=[
            pltpu.VMEM((2, EB), jnp.int32),
            pltpu.VMEM((2, EB), jnp.int32),
            pltpu.VMEM((2, EB, D), jnp.float32),
            pltpu.VMEM((2, EB, D), jnp.float32),
            pltpu.VMEM((D,), jnp.float32),
            pltpu.VMEM((n_pad,), jnp.float32),
            pltpu.VMEM_SHARED((n_pad, D), jnp.float32),
            [pltpu.SemaphoreType.DMA] * 2,
            [pltpu.SemaphoreType.DMA] * 2,
            [pltpu.SemaphoreType.DMA] * 2,
        ],
    )
    def k(xl_h, xr_h, src_h, dst_h, att_h, num_h, den_h,
          siv, div, xlg, xrg, attv, denv, acc, gsl, gsr, ssc):
        c = lax.axis_index("c")
        s = lax.axis_index("s")
        wid = s * NC + c

        pltpu.sync_copy(att_h, attv)

        zero16 = jnp.zeros((16,), jnp.float32)
        iota16 = lax.broadcasted_iota(jnp.int32, (16,), 0)
        _dn = lax.GatherDimensionNumbers(
            offset_dims=(), collapsed_slice_dims=(0,), start_index_map=(0,))

        def _shuf(v, idx):
            return lax.gather(v, idx[:, None], _dn, slice_sizes=(1,),
                              mode=lax.GatherScatterMode.PROMISE_IN_BOUNDS)

        def zden(i, carry):
            denv[pl.ds(16 * i, 16)] = zero16
            return carry

        lax.fori_loop(0, n_pad // 16, zden, 0)

        def zrow(e, carry):
            for k8 in range(D // 16):
                xlg[0, e, pl.ds(16 * k8, 16)] = zero16
            return carry

        lax.fori_loop(0, EB, zrow, 0)
        r0 = s * stripe
        for j in range(stripe // EB):
            pltpu.sync_copy(xlg.at[0], acc.at[pl.ds(r0 + j * EB, EB)])
        plsc.subcore_barrier()

        ebase = wid * tpt
        attc = [attv[pl.ds(16 * k8, 16)] for k8 in range(D // 16)]

        def start_gathers(it, bank):
            b = ebase + it * EB
            pltpu.sync_copy(src_h.at[pl.ds(b, EB)], siv.at[bank])
            pltpu.sync_copy(dst_h.at[pl.ds(b, EB)], div.at[bank])
            pltpu.async_copy(xl_h.at[siv.at[bank]], xlg.at[bank], gsl[bank])
            pltpu.async_copy(xr_h.at[div.at[bank]], xrg.at[bank], gsr[bank])

        start_gathers(0, 0)

        def outer(ot, carry):
            for bank in (0, 1):
                it = 2 * ot + bank
                nxt = it + 1
                other = 1 - bank

                @pl.when(nxt < nb)
                def _():
                    @pl.when(it >= 1)
                    def _():
                        pltpu.make_async_copy(
                            xlg.at[other], acc.at[div.at[other]],
                            ssc[other]).wait()
                    start_gathers(nxt, other)

                pltpu.make_async_copy(
                    xl_h.at[siv.at[bank]], xlg.at[bank], gsl[bank]).wait()
                pltpu.make_async_copy(
                    xr_h.at[div.at[bank]], xrg.at[bank], gsr[bank]).wait()

                b = ebase + it * EB

                def group(g, gcarry):
                    dv = div[bank, pl.ds(16 * g, 16)]
                    wacc = zero16
                    for j in range(16):
                        e = 16 * g + j
                        a = zero16
                        for k8 in range(D // 16):
                            v = (xlg[bank, e, pl.ds(16 * k8, 16)]
                                 + xrg[bank, e, pl.ds(16 * k8, 16)])
                            lv = jnp.maximum(v, 0.2 * v)
                            a = a + lv * attc[k8]
                        for sh in (1, 2, 4, 8):
                            a = a + _shuf(a, iota16 ^ sh)
                        wacc = jnp.where(iota16 == j, a, wacc)
                    mask = (b + 16 * g + iota16) < etot
                    w16 = jnp.where(mask, jnp.exp(wacc), 0.0)
                    for j in range(16):
                        e = 16 * g + j
                        w = _shuf(w16, jnp.full((16,), j, jnp.int32))
                        for k8 in range(D // 16):
                            xlg[bank, e, pl.ds(16 * k8, 16)] = (
                                xlg[bank, e, pl.ds(16 * k8, 16)] * w)
                        dval = dv[j]
                        base = dval & -16
                        lane = dval & 15
                        chunk = denv[pl.ds(base, 16)]
                        denv[pl.ds(base, 16)] = (
                            chunk + jnp.where(iota16 == lane, w, 0.0))
                    return gcarry

                lax.fori_loop(0, EB // 16, group, 0)
                pltpu.async_copy(xlg.at[bank], acc.at[div.at[bank]],
                                 ssc[bank], add=True)
            return carry

        lax.fori_loop(0, nb // 2, outer, 0)
        for bank in (0, 1):
            pltpu.make_async_copy(
                xlg.at[bank], acc.at[div.at[bank]], ssc[bank]).wait()
        plsc.subcore_barrier()

        for j in range(stripe // EB):
            rr = r0 + j * EB
            pltpu.sync_copy(acc.at[pl.ds(rr, EB)], num_h.at[c, pl.ds(rr, EB)])
        pltpu.sync_copy(denv, den_h.at[wid])

    return k(xl, xr, src, dst, att)


def _pair_gather(h, tp0, tp1, np0, np1):
    p = tp0.shape[0]
    rpt = p // NW

    mesh = plsc.VectorSubcoreMesh(core_axis_name="c", subcore_axis_name="s")

    @functools.partial(
        pl.kernel,
        out_type=(
            jax.ShapeDtypeStruct((p, 2 * D), jnp.float32),
            jax.ShapeDtypeStruct((p, 2 * D), jnp.float32),
        ),
        mesh=mesh,
        scratch_types=[
            pltpu.VMEM((rpt,), jnp.int32),
            pltpu.VMEM((rpt, D), jnp.float32),
            pltpu.SemaphoreType.DMA,
        ],
    )
    def k(h_h, tp0_h, tp1_h, np0_h, np1_h, pos_h, neg_h, iv, g, sem):
        c = lax.axis_index("c")
        s = lax.axis_index("s")
        wid = s * NC + c
        base = wid * rpt
        for idx_h, out_h, col in ((tp0_h, pos_h, 0), (tp1_h, pos_h, D),
                                  (np0_h, neg_h, 0), (np1_h, neg_h, D)):
            pltpu.sync_copy(idx_h.at[pl.ds(base, rpt)], iv)
            pltpu.async_copy(h_h.at[iv], g, sem).wait()
            pltpu.sync_copy(g, out_h.at[pl.ds(base, rpt), pl.ds(col, D)])

    return k(h, tp0, tp1, np0, np1)


def kernel(x, edge_index, true_pairs, negative_pairs,
           Wl1, Wr1, att1, b1, Wl2, Wr2, att2, b2, cW1, cb1, cW2, cb2):
    N = x.shape[0]
    E = edge_index.shape[1]
    etot = E + N

    n_pad = -(-N // (NS * EB)) * (NS * EB)
    x_pad = jnp.pad(x, ((0, n_pad - N), (0, 0)))

    ep = -(-etot // (NW * EB)) * (NW * EB)
    loop_idx = jnp.arange(N, dtype=edge_index.dtype)
    padv = jnp.zeros((ep - etot,), edge_index.dtype)
    src = jnp.concatenate([edge_index[0], loop_idx, padv])
    dst = jnp.concatenate([edge_index[1], loop_idx, padv])

    b1r = jnp.reshape(b1, (1, D))
    b2r = jnp.reshape(b2, (1, D))
    cb1r = jnp.reshape(cb1, (1, D))
    cb2r = jnp.reshape(cb2, (1, 1))
    cW2pad = jnp.pad(cW2, ((0, 0), (0, D - cW2.shape[1])))

    xl1, xr1 = _transform(x_pad, Wl1, Wr1)
    num1, den1 = _edge_pass(xl1, xr1, src, dst, att1, etot)
    xl2, xr2 = _combine_transform(num1, den1, b1r, Wl2, Wr2)

    num2, den2 = _edge_pass(xl2, xr2, src, dst, att2, etot)
    h2 = _combine(num2, den2, b2r)

    tp0 = true_pairs[:, 0]
    tp1 = true_pairs[:, 1]
    np0 = negative_pairs[:, 0]
    np1 = negative_pairs[:, 1]
    pos, neg = _pair_gather(h2, tp0, tp1, np0, np1)
    lp, ln, loss = _classifier(pos, neg, cW1, cb1r, cW2pad, cb2r)

    logits = jnp.concatenate([lp, ln], axis=0)
    return (pos, neg, logits, jnp.reshape(loss, ()))


# speedup vs baseline: 12.9538x; 1.0604x over previous
import functools

import jax
import jax.numpy as jnp
from jax import lax
from jax.experimental import pallas as pl
from jax.experimental.pallas import tpu as pltpu
from jax.experimental.pallas import tpu_sc as plsc

NC = 2
NS = 16
NW = NC * NS
EB = 64
D = 128


def _mm2_body(x_ref, wl_ref, wr_ref, xl_ref, xr_ref):
    x = x_ref[...]
    xl_ref[...] = jnp.dot(x, wl_ref[...], preferred_element_type=jnp.float32)
    xr_ref[...] = jnp.dot(x, wr_ref[...], preferred_element_type=jnp.float32)


def _transform(x, wl, wr, blk=256):
    n = x.shape[0]
    grid = n // blk
    return pl.pallas_call(
        _mm2_body,
        grid=(grid,),
        in_specs=[
            pl.BlockSpec((blk, D), lambda i: (i, 0)),
            pl.BlockSpec((D, D), lambda i: (0, 0)),
            pl.BlockSpec((D, D), lambda i: (0, 0)),
        ],
        out_specs=[
            pl.BlockSpec((blk, D), lambda i: (i, 0)),
            pl.BlockSpec((blk, D), lambda i: (i, 0)),
        ],
        out_shape=[
            jax.ShapeDtypeStruct((n, D), jnp.float32),
            jax.ShapeDtypeStruct((n, D), jnp.float32),
        ],
    )(x, wl, wr)


def _combine_mm2_body(num_ref, den_ref, b_ref, wl_ref, wr_ref, xl_ref, xr_ref):
    n = num_ref[0] + num_ref[1]
    d = jnp.sum(den_ref[...], axis=0)[:, None]
    h = jnp.maximum(n / (d + 1e-16) + b_ref[0], 0.0)
    xl_ref[...] = jnp.dot(h, wl_ref[...], preferred_element_type=jnp.float32)
    xr_ref[...] = jnp.dot(h, wr_ref[...], preferred_element_type=jnp.float32)


def _combine_transform(num, den, b, wl, wr, blk=256):
    n = num.shape[1]
    grid = n // blk
    return pl.pallas_call(
        _combine_mm2_body,
        grid=(grid,),
        in_specs=[
            pl.BlockSpec((NC, blk, D), lambda i: (0, i, 0)),
            pl.BlockSpec((NW, blk), lambda i: (0, i)),
            pl.BlockSpec((1, D), lambda i: (0, 0)),
            pl.BlockSpec((D, D), lambda i: (0, 0)),
            pl.BlockSpec((D, D), lambda i: (0, 0)),
        ],
        out_specs=[
            pl.BlockSpec((blk, D), lambda i: (i, 0)),
            pl.BlockSpec((blk, D), lambda i: (i, 0)),
        ],
        out_shape=[
            jax.ShapeDtypeStruct((n, D), jnp.float32),
            jax.ShapeDtypeStruct((n, D), jnp.float32),
        ],
    )(num, den, b, wl, wr)


def _combine_body(num_ref, den_ref, b_ref, h_ref):
    n = num_ref[0] + num_ref[1]
    d = jnp.sum(den_ref[...], axis=0)[:, None]
    h_ref[...] = n / (d + 1e-16) + b_ref[0]


def _combine(num, den, b, blk=256):
    n = num.shape[1]
    grid = n // blk
    return pl.pallas_call(
        _combine_body,
        grid=(grid,),
        in_specs=[
            pl.BlockSpec((NC, blk, D), lambda i: (0, i, 0)),
            pl.BlockSpec((NW, blk), lambda i: (0, i)),
            pl.BlockSpec((1, D), lambda i: (0, 0)),
        ],
        out_specs=pl.BlockSpec((blk, D), lambda i: (i, 0)),
        out_shape=jax.ShapeDtypeStruct((n, D), jnp.float32),
    )(num, den, b)


def _classifier_body(pos_ref, neg_ref, w1_ref, b1_ref, w2_ref, b2_ref,
                     lp_ref, ln_ref, loss_ref, acc_ref):
    i = pl.program_id(0)
    nsteps = pl.num_programs(0)

    @pl.when(i == 0)
    def _():
        acc_ref[0] = 0.0

    w1 = w1_ref[...]
    w2 = w2_ref[...]
    b1 = b1_ref[0]
    b2 = b2_ref[0, 0]

    zp = jnp.maximum(jnp.dot(pos_ref[...], w1, preferred_element_type=jnp.float32) + b1, 0.0)
    zn = jnp.maximum(jnp.dot(neg_ref[...], w1, preferred_element_type=jnp.float32) + b1, 0.0)
    lop = jnp.dot(zp, w2, preferred_element_type=jnp.float32)[:, 0:1] + b2
    lon = jnp.dot(zn, w2, preferred_element_type=jnp.float32)[:, 0:1] + b2
    sp = jax.nn.sigmoid(lop)
    sn = jax.nn.sigmoid(lon)
    lp_ref[...] = sp
    ln_ref[...] = sn
    pp = jnp.clip(sp, 1e-7, 1.0 - 1e-7)
    pn = jnp.clip(sn, 1e-7, 1.0 - 1e-7)
    acc_ref[0] += jnp.sum(jnp.log(pp)) + jnp.sum(jnp.log(1.0 - pn))

    @pl.when(i == nsteps - 1)
    def _():
        loss_ref[...] = (-acc_ref[0] / (2.0 * ln_ref.shape[0] * nsteps)
                         ) * jnp.ones((1, 1), jnp.float32)


def _classifier(pos, neg, w1, b1, w2pad, b2, blk=512):
    p = pos.shape[0]
    grid = p // blk
    return pl.pallas_call(
        _classifier_body,
        grid=(grid,),
        in_specs=[
            pl.BlockSpec((blk, 2 * D), lambda i: (i, 0)),
            pl.BlockSpec((blk, 2 * D), lambda i: (i, 0)),
            pl.BlockSpec((2 * D, D), lambda i: (0, 0)),
            pl.BlockSpec((1, D), lambda i: (0, 0)),
            pl.BlockSpec((D, D), lambda i: (0, 0)),
            pl.BlockSpec((1, 1), lambda i: (0, 0)),
        ],
        out_specs=[
            pl.BlockSpec((blk, 1), lambda i: (i, 0)),
            pl.BlockSpec((blk, 1), lambda i: (i, 0)),
            pl.BlockSpec((1, 1), lambda i: (0, 0)),
        ],
        out_shape=[
            jax.ShapeDtypeStruct((p, 1), jnp.float32),
            jax.ShapeDtypeStruct((p, 1), jnp.float32),
            jax.ShapeDtypeStruct((1, 1), jnp.float32),
        ],
        scratch_shapes=[pltpu.SMEM((1,), jnp.float32)],
    )(pos, neg, w1, b1, w2pad, b2)


def _edge_pass(xl, xr, src, dst, att, etot):
    n_pad = xl.shape[0]
    ep = src.shape[0]
    tpt = ep // NW
    nb = tpt // EB
    stripe = n_pad // NS

    mesh = plsc.VectorSubcoreMesh(core_axis_name="c", subcore_axis_name="s")

    @functools.partial(
        pl.kernel,
        out_type=(
            jax.ShapeDtypeStruct((NC, n_pad, D), jnp.float32),
            jax.ShapeDtypeStruct((NW, n_pad), jnp.float32),
        ),
        mesh=mesh,
        scratch_types=[
            pltpu.VMEM((4, EB), jnp.int32),
            pltpu.VMEM((4, EB), jnp.int32),
            pltpu.VMEM((2, EB, D), jnp.float32),
            pltpu.VMEM((2, EB, D), jnp.float32),
            pltpu.VMEM((D,), jnp.float32),
            pltpu.VMEM((n_pad,), jnp.float32),
            pltpu.VMEM_SHARED((n_pad, D), jnp.float32),
            [pltpu.SemaphoreType.DMA] * 2,
            [pltpu.SemaphoreType.DMA] * 2,
            [pltpu.SemaphoreType.DMA] * 2,
            [pltpu.SemaphoreType.DMA] * 4,
            [pltpu.SemaphoreType.DMA] * 4,
        ],
    )
    def k(xl_h, xr_h, src_h, dst_h, att_h, num_h, den_h,
          siv, div, xlg, xrg, attv, denv, acc, gsl, gsr, ssc, isl, isr):
        c = lax.axis_index("c")
        s = lax.axis_index("s")
        wid = s * NC + c

        pltpu.sync_copy(att_h, attv)

        zero16 = jnp.zeros((16,), jnp.float32)
        iota16 = lax.broadcasted_iota(jnp.int32, (16,), 0)
        _dn = lax.GatherDimensionNumbers(
            offset_dims=(), collapsed_slice_dims=(0,), start_index_map=(0,))

        def _shuf(v, idx):
            return lax.gather(v, idx[:, None], _dn, slice_sizes=(1,),
                              mode=lax.GatherScatterMode.PROMISE_IN_BOUNDS)

        def zden(i, carry):
            denv[pl.ds(16 * i, 16)] = zero16
            return carry

        lax.fori_loop(0, n_pad // 16, zden, 0)

        def zrow(e, carry):
            for k8 in range(D // 16):
                xlg[0, e, pl.ds(16 * k8, 16)] = zero16
            return carry

        lax.fori_loop(0, EB, zrow, 0)
        r0 = s * stripe
        for j in range(stripe // EB):
            pltpu.sync_copy(xlg.at[0], acc.at[pl.ds(r0 + j * EB, EB)])
        plsc.subcore_barrier()

        ebase = wid * tpt
        attc = [attv[pl.ds(16 * k8, 16)] for k8 in range(D // 16)]

        def start_gathers(it, b2, b4):
            pltpu.async_copy(xl_h.at[siv.at[b4]], xlg.at[b2], gsl[b2])
            pltpu.async_copy(xr_h.at[div.at[b4]], xrg.at[b2], gsr[b2])

        pltpu.sync_copy(src_h.at[pl.ds(ebase, EB)], siv.at[0])
        pltpu.sync_copy(dst_h.at[pl.ds(ebase, EB)], div.at[0])
        pltpu.sync_copy(src_h.at[pl.ds(ebase + EB, EB)], siv.at[1])
        pltpu.sync_copy(dst_h.at[pl.ds(ebase + EB, EB)], div.at[1])
        start_gathers(0, 0, 0)

        def outer(ot, carry):
            for sub in range(4):
                it = 4 * ot + sub
                nxt = it + 1
                bank = sub % 2
                other = 1 - bank
                ib2 = (sub + 2) % 4
                ib1 = (sub + 1) % 4

                @pl.when(it + 2 < nb)
                def _():
                    bi = ebase + (it + 2) * EB
                    pltpu.async_copy(src_h.at[pl.ds(bi, EB)], siv.at[ib2],
                                     isl[ib2])
                    pltpu.async_copy(dst_h.at[pl.ds(bi, EB)], div.at[ib2],
                                     isr[ib2])

                @pl.when(nxt < nb)
                def _():
                    @pl.when(it >= 1)
                    def _():
                        pltpu.make_async_copy(
                            xlg.at[other], acc.at[div.at[other]],
                            ssc[other]).wait()

                    @pl.when(nxt >= 2)
                    def _():
                        bi = ebase + nxt * EB
                        pltpu.make_async_copy(
                            src_h.at[pl.ds(bi, EB)], siv.at[ib1],
                            isl[ib1]).wait()
                        pltpu.make_async_copy(
                            dst_h.at[pl.ds(bi, EB)], div.at[ib1],
                            isr[ib1]).wait()
                    start_gathers(nxt, other, ib1)

                pltpu.make_async_copy(
                    xl_h.at[siv.at[sub]], xlg.at[bank], gsl[bank]).wait()
                pltpu.make_async_copy(
                    xr_h.at[div.at[sub]], xrg.at[bank], gsr[bank]).wait()

                b = ebase + it * EB

                def group(g, gcarry):
                    dv = div[sub, pl.ds(16 * g, 16)]
                    wacc = zero16
                    for j in range(16):
                        e = 16 * g + j
                        a = zero16
                        for k8 in range(D // 16):
                            v = (xlg[bank, e, pl.ds(16 * k8, 16)]
                                 + xrg[bank, e, pl.ds(16 * k8, 16)])
                            lv = jnp.maximum(v, 0.2 * v)
                            a = a + lv * attc[k8]
                        for sh in (1, 2, 4, 8):
                            a = a + _shuf(a, iota16 ^ sh)
                        wacc = jnp.where(iota16 == j, a, wacc)
                    mask = (b + 16 * g + iota16) < etot
                    w16 = jnp.where(mask, jnp.exp(wacc), 0.0)
                    for j in range(16):
                        e = 16 * g + j
                        w = _shuf(w16, jnp.full((16,), j, jnp.int32))
                        for k8 in range(D // 16):
                            xlg[bank, e, pl.ds(16 * k8, 16)] = (
                                xlg[bank, e, pl.ds(16 * k8, 16)] * w)
                        dval = dv[j]
                        base = dval & -16
                        lane = dval & 15
                        chunk = denv[pl.ds(base, 16)]
                        denv[pl.ds(base, 16)] = (
                            chunk + jnp.where(iota16 == lane, w, 0.0))
                    return gcarry

                lax.fori_loop(0, EB // 16, group, 0)
                pltpu.async_copy(xlg.at[bank], acc.at[div.at[sub]],
                                 ssc[bank], add=True)
            return carry

        lax.fori_loop(0, nb // 4, outer, 0)
        for bank in (0, 1):
            pltpu.make_async_copy(
                xlg.at[bank], acc.at[div.at[bank]], ssc[bank]).wait()
        plsc.subcore_barrier()

        for j in range(stripe // EB):
            rr = r0 + j * EB
            pltpu.sync_copy(acc.at[pl.ds(rr, EB)], num_h.at[c, pl.ds(rr, EB)])
        pltpu.sync_copy(denv, den_h.at[wid])

    return k(xl, xr, src, dst, att)


def _pair_gather(h, tp0, tp1, np0, np1):
    p = tp0.shape[0]
    rpt = p // NW

    mesh = plsc.VectorSubcoreMesh(core_axis_name="c", subcore_axis_name="s")

    @functools.partial(
        pl.kernel,
        out_type=(
            jax.ShapeDtypeStruct((p, 2 * D), jnp.float32),
            jax.ShapeDtypeStruct((p, 2 * D), jnp.float32),
        ),
        mesh=mesh,
        scratch_types=[
            pltpu.VMEM((rpt,), jnp.int32),
            pltpu.VMEM((rpt, D), jnp.float32),
            pltpu.SemaphoreType.DMA,
        ],
    )
    def k(h_h, tp0_h, tp1_h, np0_h, np1_h, pos_h, neg_h, iv, g, sem):
        c = lax.axis_index("c")
        s = lax.axis_index("s")
        wid = s * NC + c
        base = wid * rpt
        for idx_h, out_h, col in ((tp0_h, pos_h, 0), (tp1_h, pos_h, D),
                                  (np0_h, neg_h, 0), (np1_h, neg_h, D)):
            pltpu.sync_copy(idx_h.at[pl.ds(base, rpt)], iv)
            pltpu.async_copy(h_h.at[iv], g, sem).wait()
            pltpu.sync_copy(g, out_h.at[pl.ds(base, rpt), pl.ds(col, D)])

    return k(h, tp0, tp1, np0, np1)


def kernel(x, edge_index, true_pairs, negative_pairs,
           Wl1, Wr1, att1, b1, Wl2, Wr2, att2, b2, cW1, cb1, cW2, cb2):
    N = x.shape[0]
    E = edge_index.shape[1]
    etot = E + N

    n_pad = -(-N // (NS * EB)) * (NS * EB)
    x_pad = jnp.pad(x, ((0, n_pad - N), (0, 0)))

    ep = -(-etot // (NW * EB * 4)) * (NW * EB * 4)
    loop_idx = jnp.arange(N, dtype=edge_index.dtype)
    padv = jnp.zeros((ep - etot,), edge_index.dtype)
    src = jnp.concatenate([edge_index[0], loop_idx, padv])
    dst = jnp.concatenate([edge_index[1], loop_idx, padv])

    b1r = jnp.reshape(b1, (1, D))
    b2r = jnp.reshape(b2, (1, D))
    cb1r = jnp.reshape(cb1, (1, D))
    cb2r = jnp.reshape(cb2, (1, 1))
    cW2pad = jnp.pad(cW2, ((0, 0), (0, D - cW2.shape[1])))

    xl1, xr1 = _transform(x_pad, Wl1, Wr1)
    num1, den1 = _edge_pass(xl1, xr1, src, dst, att1, etot)
    xl2, xr2 = _combine_transform(num1, den1, b1r, Wl2, Wr2)

    num2, den2 = _edge_pass(xl2, xr2, src, dst, att2, etot)
    h2 = _combine(num2, den2, b2r)

    tp0 = true_pairs[:, 0]
    tp1 = true_pairs[:, 1]
    np0 = negative_pairs[:, 0]
    np1 = negative_pairs[:, 1]
    pos, neg = _pair_gather(h2, tp0, tp1, np0, np1)
    lp, ln, loss = _classifier(pos, neg, cW1, cb1r, cW2pad, cb2r)

    logits = jnp.concatenate([lp, ln], axis=0)
    return (pos, neg, logits, jnp.reshape(loss, ()))
